# baseline (speedup 1.0000x reference)
import functools

import jax
import jax.numpy as jnp
from jax import lax
from jax.experimental import pallas as pl
from jax.experimental.pallas import tpu as pltpu
from jax.experimental.pallas import tpu_sc as plsc

B = 16384
D = 128
NC = 2
NS = 16
L = 16
NW = NC * NS
BPW = B // NW
CH = 64
NCHUNK = BPW // CH
NBUF = 3


def _build():
    mesh = plsc.VectorSubcoreMesh(core_axis_name="c", subcore_axis_name="s")

    @functools.partial(
        pl.kernel,
        out_type=jax.ShapeDtypeStruct((B,), jnp.float32),
        mesh=mesh,
        scratch_types=[
            pltpu.VMEM((2, NCHUNK, CH), jnp.int32),
            pltpu.VMEM((2 * NBUF, CH, D), jnp.float32),
            pltpu.VMEM((BPW,), jnp.float32),
            pltpu.SemaphoreType.DMA,
            pltpu.SemaphoreType.DMA,
            pltpu.SemaphoreType.DMA,
        ],
        compiler_params=pltpu.CompilerParams(needs_layout_passes=False),
    )
    def scored(idx_hbm, uemb_hbm, iemb_hbm, out_hbm,
               idx_v, rows_v, out_v, sem0, sem1, sem2):
        wid = lax.axis_index("s") * NC + lax.axis_index("c")
        pltpu.sync_copy(idx_hbm.at[0, wid], idx_v.at[0])
        pltpu.sync_copy(idx_hbm.at[1, wid], idx_v.at[1])
        lanes = lax.iota(jnp.int32, L)
        sems = (sem0, sem1, sem2)

        def start(c):
            b = c % NBUF
            return (pltpu.async_copy(uemb_hbm.at[idx_v.at[0, c]],
                                     rows_v.at[2 * b], sems[b]),
                    pltpu.async_copy(iemb_hbm.at[idx_v.at[1, c]],
                                     rows_v.at[2 * b + 1], sems[b]))

        pend = [start(0), start(1)]
        for c in range(NCHUNK):
            if c + 2 < NCHUNK:
                pend.append(start(c + 2))
            pend[c][0].wait()
            pend[c][1].wait()
            b = c % NBUF
            ub, ib = rows_v.at[2 * b], rows_v.at[2 * b + 1]
            for g in range(CH // L):
                rows16 = lanes + (g * L)

                @plsc.parallel_loop(0, D, unroll=8,
                                    carry=jnp.zeros((L,), jnp.float32))
                def acc(dd, acc_in):
                    dvec = (lanes + dd) & (D - 1)
                    u = plsc.load_gather(ub, [rows16, dvec])
                    it = plsc.load_gather(ib, [rows16, dvec])
                    return acc_in + u * it

                out_v[pl.ds(c * CH + g * L, L)] = acc
        pltpu.sync_copy(out_v, out_hbm.at[pl.ds(wid * BPW, BPW)])

    return scored


_scored = _build()


def kernel(user_list, item_list, user_embeddings, item_embeddings):
    idx = jnp.stack(
        [user_list.astype(jnp.int32), item_list.astype(jnp.int32)]
    ).reshape(2, NW, NCHUNK, CH)
    return _scored(idx, user_embeddings, item_embeddings)
